# Initial kernel scaffold; baseline (speedup 1.0000x reference)
import jax
import jax.numpy as jnp
from jax.experimental import pallas as pl


def kernel(x, edge_index, W1l, b1l, W1r, W2l, b2l, W2r):
    raise NotImplementedError("write your pallas kernel here")


# speedup vs baseline: 5.6702x; 5.6702x over previous
import functools

import jax
import jax.numpy as jnp
from jax import lax
from jax.experimental import pallas as pl
from jax.experimental.pallas import tpu as pltpu
from jax.experimental.pallas import tpu_sc as plsc

NC = 2
NS = 16
NW = NC * NS
CHUNK = 128
ROWBLK = 1024
DPAD = 16


def _sc_agg_kernel(nr, da, nchunk):
    rows_per_tile = nr // NS
    nzc = rows_per_tile // CHUNK
    assert rows_per_tile % CHUNK == 0
    mesh = plsc.VectorSubcoreMesh(
        core_axis_name="c", subcore_axis_name="s", num_cores=NC, num_subcores=NS
    )
    scratch = [
        pltpu.VMEM((CHUNK,), jnp.int32),
        pltpu.VMEM((CHUNK,), jnp.int32),
        pltpu.VMEM((CHUNK,), jnp.int32),
        pltpu.VMEM((CHUNK, da), jnp.float32),
        pltpu.VMEM_SHARED((nr, da), jnp.float32),
        pltpu.SemaphoreType.DMA,
    ]

    def body(feats, src_hbm, dst_hbm, wb_hbm, zf_hbm,
             sum_out, src_v, dst_v, wb_v, rows_v, sum_sh, sem):
        c = lax.axis_index("c")
        s = lax.axis_index("s")
        wid = c * NS + s
        row0 = s * rows_per_tile
        out0 = c * nr + row0

        pltpu.sync_copy(zf_hbm, rows_v)
        for k in range(nzc):
            pltpu.sync_copy(wb_hbm.at[pl.ds(row0 + k * CHUNK, CHUNK)], wb_v)
            pltpu.sync_copy(rows_v, sum_sh.at[wb_v])
        plsc.subcore_barrier()

        def chunk_body(j, carry):
            pltpu.sync_copy(src_hbm.at[wid, j], src_v)
            pltpu.sync_copy(dst_hbm.at[wid, j], dst_v)
            pltpu.async_copy(feats.at[src_v], rows_v, sem).wait()
            pltpu.sync_copy(rows_v, sum_sh.at[dst_v], add=True)
            return carry

        lax.fori_loop(0, nchunk, chunk_body, 0)
        plsc.subcore_barrier()

        for k in range(nzc):
            pltpu.sync_copy(wb_hbm.at[pl.ds(row0 + k * CHUNK, CHUNK)], wb_v)
            pltpu.async_copy(sum_sh.at[wb_v], rows_v, sem).wait()
            pltpu.sync_copy(rows_v, sum_out.at[pl.ds(out0 + k * CHUNK, CHUNK)])

    return pl.kernel(
        body,
        out_type=jax.ShapeDtypeStruct((NC * nr, da), jnp.float32),
        mesh=mesh,
        scratch_types=scratch,
        compiler_params=pltpu.CompilerParams(use_tc_tiling_on_sc=False),
    )


def _tc_layer_kernel(relu, d, p_ref, x_ref, wlt_ref, bl_ref, wrt_ref, o_ref):
    psum = p_ref[0] + p_ref[1]
    summed = psum[:, :d]
    deg = psum[:, d:d + 1]
    mean = summed / jnp.maximum(deg, 1.0)
    h = (jnp.dot(mean, wlt_ref[...], preferred_element_type=jnp.float32)
         + bl_ref[...]
         + jnp.dot(x_ref[...], wrt_ref[...], preferred_element_type=jnp.float32))
    nrm = jnp.sqrt(jnp.sum(h * h, axis=1, keepdims=True))
    h = h / jnp.maximum(nrm, 1e-12)
    if relu:
        h = jnp.maximum(h, 0.0)
    o_ref[...] = h


def _tc_layer(relu, nr, d, parts, x, wlt, bl, wrt):
    nblk = nr // ROWBLK
    da = d + DPAD
    return pl.pallas_call(
        functools.partial(_tc_layer_kernel, relu, d),
        grid=(nblk,),
        in_specs=[
            pl.BlockSpec((NC, ROWBLK, da), lambda b: (0, b, 0)),
            pl.BlockSpec((ROWBLK, d), lambda b: (b, 0)),
            pl.BlockSpec((d, d), lambda b: (0, 0)),
            pl.BlockSpec((1, d), lambda b: (0, 0)),
            pl.BlockSpec((d, d), lambda b: (0, 0)),
        ],
        out_specs=pl.BlockSpec((ROWBLK, d), lambda b: (b, 0)),
        out_shape=jax.ShapeDtypeStruct((nr, d), jnp.float32),
    )(parts, x, wlt, bl, wrt)


def kernel(x, edge_index, W1l, b1l, W1r, W2l, b2l, W2r):
    n, d = x.shape
    da = d + DPAD
    e = edge_index.shape[1]
    nr = ((n + ROWBLK - 1) // ROWBLK) * ROWBLK
    epw_chunks = -(-e // (NW * CHUNK))
    ep = NW * CHUNK * epw_chunks

    src = edge_index[0].astype(jnp.int32)
    dst = edge_index[1].astype(jnp.int32)
    pad = ep - e
    pad_src = jnp.arange(pad, dtype=jnp.int32) % n
    pad_dst = n + jnp.arange(pad, dtype=jnp.int32) % (nr - n)
    src_p = jnp.concatenate([src, pad_src]).reshape(NW, epw_chunks, CHUNK)
    dst_p = jnp.concatenate([dst, pad_dst]).reshape(NW, epw_chunks, CHUNK)
    wb = jnp.arange(nr, dtype=jnp.int32)

    onescol = jnp.ones((nr, DPAD), jnp.float32)
    x_p = jnp.zeros((nr, d), jnp.float32).at[:n].set(x)
    x_aug = jnp.concatenate([x_p, onescol], axis=1)
    zf = jnp.zeros((CHUNK, da), jnp.float32)
    wl1t, wr1t = W1l.T, W1r.T
    wl2t, wr2t = W2l.T, W2r.T
    bl1 = b1l.reshape(1, d)
    bl2 = b2l.reshape(1, d)

    agg = _sc_agg_kernel(nr, da, epw_chunks)

    p1 = agg(x_aug, src_p, dst_p, wb, zf).reshape(NC, nr, da)
    h = _tc_layer(True, nr, d, p1, x_p, wl1t, bl1, wr1t)
    h_aug = jnp.concatenate([h, onescol], axis=1)
    p2 = agg(h_aug, src_p, dst_p, wb, zf).reshape(NC, nr, da)
    out = _tc_layer(False, nr, d, p2, h, wl2t, bl2, wr2t)
    return out[:n]


# speedup vs baseline: 7.6703x; 1.3527x over previous
import functools

import jax
import jax.numpy as jnp
from jax import lax
from jax.experimental import pallas as pl
from jax.experimental.pallas import tpu as pltpu
from jax.experimental.pallas import tpu_sc as plsc

NC = 2
NS = 16
NW = NC * NS
CHUNK = 64
GB = 8
ROWBLK = 1024
DPAD = 16
WBC = 128


def _sc_agg_kernel(nr, da, nbatch):
    rows_per_tile = nr // NS
    nzc = rows_per_tile // WBC
    assert rows_per_tile % WBC == 0
    mesh = plsc.VectorSubcoreMesh(
        core_axis_name="c", subcore_axis_name="s", num_cores=NC, num_subcores=NS
    )
    scratch = [
        pltpu.VMEM((GB, CHUNK), jnp.int32),
        pltpu.VMEM((GB, CHUNK), jnp.int32),
        pltpu.VMEM((WBC,), jnp.int32),
        pltpu.VMEM((CHUNK, da), jnp.float32),
        pltpu.VMEM((CHUNK, da), jnp.float32),
        pltpu.VMEM((WBC, da), jnp.float32),
        pltpu.VMEM_SHARED((nr, da), jnp.float32),
        pltpu.SemaphoreType.DMA,
        pltpu.SemaphoreType.DMA,
    ]

    def body(feats, src_hbm, dst_hbm, wb_hbm, zf_hbm,
             sum_out, src_b, dst_b, wb_v, rows0, rows1, stage_v, sum_sh,
             sem0, sem1):
        c = lax.axis_index("c")
        s = lax.axis_index("s")
        wid = c * NS + s
        row0 = s * rows_per_tile
        out0 = c * nr + row0
        bufs = (rows0, rows1)
        sems = (sem0, sem1)

        pltpu.sync_copy(zf_hbm, stage_v)
        for k in range(nzc):
            pltpu.sync_copy(wb_hbm.at[pl.ds(row0 + k * WBC, WBC)], wb_v)
            pltpu.sync_copy(stage_v, sum_sh.at[wb_v])
        plsc.subcore_barrier()

        def batch_body(b, carry):
            pltpu.sync_copy(src_hbm.at[wid * nbatch + b], src_b)
            pltpu.sync_copy(dst_hbm.at[wid * nbatch + b], dst_b)
            descs = [None, None]
            descs[0] = pltpu.async_copy(feats.at[src_b.at[0]], bufs[0],
                                        sems[0])
            for g in range(GB):
                cur = g % 2
                if g + 1 < GB:
                    nxt = 1 - cur
                    descs[nxt] = pltpu.async_copy(
                        feats.at[src_b.at[g + 1]], bufs[nxt], sems[nxt])
                descs[cur].wait()
                pltpu.sync_copy(bufs[cur], sum_sh.at[dst_b.at[g]], add=True)
            return carry

        lax.fori_loop(0, nbatch, batch_body, 0)
        plsc.subcore_barrier()

        for k in range(nzc):
            pltpu.sync_copy(wb_hbm.at[pl.ds(row0 + k * WBC, WBC)], wb_v)
            pltpu.async_copy(sum_sh.at[wb_v], stage_v, sem0).wait()
            pltpu.sync_copy(stage_v, sum_out.at[pl.ds(out0 + k * WBC, WBC)])

    return pl.kernel(
        body,
        out_type=jax.ShapeDtypeStruct((NC * nr, da), jnp.float32),
        mesh=mesh,
        scratch_types=scratch,
        compiler_params=pltpu.CompilerParams(use_tc_tiling_on_sc=False),
    )


def _tc_layer_kernel(relu, d, p_ref, x_ref, wlt_ref, bl_ref, wrt_ref, o_ref):
    psum = p_ref[0] + p_ref[1]
    summed = psum[:, :d]
    deg = psum[:, d:d + 1]
    mean = summed / jnp.maximum(deg, 1.0)
    h = (jnp.dot(mean, wlt_ref[...], preferred_element_type=jnp.float32)
         + bl_ref[...]
         + jnp.dot(x_ref[...], wrt_ref[...], preferred_element_type=jnp.float32))
    nrm = jnp.sqrt(jnp.sum(h * h, axis=1, keepdims=True))
    h = h / jnp.maximum(nrm, 1e-12)
    if relu:
        h = jnp.maximum(h, 0.0)
    o_ref[...] = h


def _tc_layer(relu, nr, d, parts, x, wlt, bl, wrt):
    nblk = nr // ROWBLK
    da = d + DPAD
    return pl.pallas_call(
        functools.partial(_tc_layer_kernel, relu, d),
        grid=(nblk,),
        in_specs=[
            pl.BlockSpec((NC, ROWBLK, da), lambda b: (0, b, 0)),
            pl.BlockSpec((ROWBLK, d), lambda b: (b, 0)),
            pl.BlockSpec((d, d), lambda b: (0, 0)),
            pl.BlockSpec((1, d), lambda b: (0, 0)),
            pl.BlockSpec((d, d), lambda b: (0, 0)),
        ],
        out_specs=pl.BlockSpec((ROWBLK, d), lambda b: (b, 0)),
        out_shape=jax.ShapeDtypeStruct((nr, d), jnp.float32),
    )(parts, x, wlt, bl, wrt)


def kernel(x, edge_index, W1l, b1l, W1r, W2l, b2l, W2r):
    n, d = x.shape
    da = d + DPAD
    e = edge_index.shape[1]
    nr = ((n + ROWBLK - 1) // ROWBLK) * ROWBLK
    unit = NW * GB * CHUNK
    nbatch = -(-e // unit)
    ep = unit * nbatch

    src = edge_index[0].astype(jnp.int32)
    dst = edge_index[1].astype(jnp.int32)
    pad = ep - e
    pad_src = jnp.arange(pad, dtype=jnp.int32) % n
    pad_dst = n + jnp.arange(pad, dtype=jnp.int32) % (nr - n)
    src_p = jnp.concatenate([src, pad_src]).reshape(NW * nbatch, GB, CHUNK)
    dst_p = jnp.concatenate([dst, pad_dst]).reshape(NW * nbatch, GB, CHUNK)
    wb = jnp.arange(nr, dtype=jnp.int32)

    onescol = jnp.ones((nr, DPAD), jnp.float32)
    x_p = jnp.zeros((nr, d), jnp.float32).at[:n].set(x)
    x_aug = jnp.concatenate([x_p, onescol], axis=1)
    zf = jnp.zeros((WBC, da), jnp.float32)
    wl1t, wr1t = W1l.T, W1r.T
    wl2t, wr2t = W2l.T, W2r.T
    bl1 = b1l.reshape(1, d)
    bl2 = b2l.reshape(1, d)

    agg = _sc_agg_kernel(nr, da, nbatch)

    p1 = agg(x_aug, src_p, dst_p, wb, zf).reshape(NC, nr, da)
    h = _tc_layer(True, nr, d, p1, x_p, wl1t, bl1, wr1t)
    h_aug = jnp.concatenate([h, onescol], axis=1)
    p2 = agg(h_aug, src_p, dst_p, wb, zf).reshape(NC, nr, da)
    out = _tc_layer(False, nr, d, p2, h, wl2t, bl2, wr2t)
    return out[:n]


# speedup vs baseline: 7.8256x; 1.0202x over previous
import functools

import jax
import jax.numpy as jnp
from jax import lax
from jax.experimental import pallas as pl
from jax.experimental.pallas import tpu as pltpu
from jax.experimental.pallas import tpu_sc as plsc

NC = 2
NS = 16
NW = NC * NS
CHUNK = 128
GB = 4
ROWBLK = 1024
DPAD = 16
WBC = 128


def _sc_agg_kernel(nr, da, nbatch):
    rows_per_tile = nr // NS
    nzc = rows_per_tile // WBC
    assert rows_per_tile % WBC == 0
    mesh = plsc.VectorSubcoreMesh(
        core_axis_name="c", subcore_axis_name="s", num_cores=NC, num_subcores=NS
    )
    scratch = [
        pltpu.VMEM((GB, CHUNK), jnp.int32),
        pltpu.VMEM((GB, CHUNK), jnp.int32),
        pltpu.VMEM((CHUNK, da), jnp.float32),
        pltpu.VMEM((WBC, da), jnp.float32),
        pltpu.VMEM_SHARED((nr, da), jnp.float32),
        pltpu.SemaphoreType.DMA,
        pltpu.SemaphoreType.DMA,
    ]

    def body(feats, src_hbm, dst_hbm, wb_hbm, zf_hbm,
             sum_out, src_b, dst_b, rows0, stage_v, sum_sh,
             sem0, sem1):
        c = lax.axis_index("c")
        s = lax.axis_index("s")
        wid = c * NS + s
        row0 = s * rows_per_tile
        out0 = c * nr + row0
        bufs = (rows0, stage_v)
        sems = (sem0, sem1)

        wbrow = src_b.at[0]
        pltpu.sync_copy(zf_hbm, stage_v)
        for k in range(nzc):
            pltpu.sync_copy(wb_hbm.at[pl.ds(row0 + k * WBC, WBC)], wbrow)
            pltpu.sync_copy(stage_v, sum_sh.at[wbrow])
        plsc.subcore_barrier()

        def batch_body(b, carry):
            pltpu.sync_copy(src_hbm.at[wid * nbatch + b], src_b)
            pltpu.sync_copy(dst_hbm.at[wid * nbatch + b], dst_b)
            descs = [None, None]
            descs[0] = pltpu.async_copy(feats.at[src_b.at[0]], bufs[0],
                                        sems[0])
            for g in range(GB):
                cur = g % 2
                if g + 1 < GB:
                    nxt = 1 - cur
                    descs[nxt] = pltpu.async_copy(
                        feats.at[src_b.at[g + 1]], bufs[nxt], sems[nxt])
                descs[cur].wait()
                pltpu.sync_copy(bufs[cur], sum_sh.at[dst_b.at[g]], add=True)
            return carry

        lax.fori_loop(0, nbatch, batch_body, 0)
        plsc.subcore_barrier()

        for k in range(nzc):
            pltpu.sync_copy(wb_hbm.at[pl.ds(row0 + k * WBC, WBC)], wbrow)
            pltpu.async_copy(sum_sh.at[wbrow], stage_v, sem0).wait()
            pltpu.sync_copy(stage_v, sum_out.at[pl.ds(out0 + k * WBC, WBC)])

    return pl.kernel(
        body,
        out_type=jax.ShapeDtypeStruct((NC * nr, da), jnp.float32),
        mesh=mesh,
        scratch_types=scratch,
        compiler_params=pltpu.CompilerParams(use_tc_tiling_on_sc=False),
    )


def _tc_layer_kernel(relu, d, p_ref, x_ref, wlt_ref, bl_ref, wrt_ref, o_ref):
    psum = p_ref[0] + p_ref[1]
    summed = psum[:, :d]
    deg = psum[:, d:d + 1]
    mean = summed / jnp.maximum(deg, 1.0)
    h = (jnp.dot(mean, wlt_ref[...], preferred_element_type=jnp.float32)
         + bl_ref[...]
         + jnp.dot(x_ref[...], wrt_ref[...], preferred_element_type=jnp.float32))
    nrm = jnp.sqrt(jnp.sum(h * h, axis=1, keepdims=True))
    h = h / jnp.maximum(nrm, 1e-12)
    if relu:
        h = jnp.maximum(h, 0.0)
    o_ref[...] = h


def _tc_layer(relu, nr, d, parts, x, wlt, bl, wrt):
    nblk = nr // ROWBLK
    da = d + DPAD
    return pl.pallas_call(
        functools.partial(_tc_layer_kernel, relu, d),
        grid=(nblk,),
        in_specs=[
            pl.BlockSpec((NC, ROWBLK, da), lambda b: (0, b, 0)),
            pl.BlockSpec((ROWBLK, d), lambda b: (b, 0)),
            pl.BlockSpec((d, d), lambda b: (0, 0)),
            pl.BlockSpec((1, d), lambda b: (0, 0)),
            pl.BlockSpec((d, d), lambda b: (0, 0)),
        ],
        out_specs=pl.BlockSpec((ROWBLK, d), lambda b: (b, 0)),
        out_shape=jax.ShapeDtypeStruct((nr, d), jnp.float32),
    )(parts, x, wlt, bl, wrt)


def kernel(x, edge_index, W1l, b1l, W1r, W2l, b2l, W2r):
    n, d = x.shape
    da = d + DPAD
    e = edge_index.shape[1]
    nr = ((n + ROWBLK - 1) // ROWBLK) * ROWBLK
    unit = NW * GB * CHUNK
    nbatch = -(-e // unit)
    ep = unit * nbatch

    src = edge_index[0].astype(jnp.int32)
    dst = edge_index[1].astype(jnp.int32)
    pad = ep - e
    pad_src = jnp.arange(pad, dtype=jnp.int32) % n
    pad_dst = n + jnp.arange(pad, dtype=jnp.int32) % (nr - n)
    src_p = jnp.concatenate([src, pad_src]).reshape(NW * nbatch, GB, CHUNK)
    dst_p = jnp.concatenate([dst, pad_dst]).reshape(NW * nbatch, GB, CHUNK)
    wb = jnp.arange(nr, dtype=jnp.int32)

    onescol = jnp.ones((nr, DPAD), jnp.float32)
    x_p = jnp.zeros((nr, d), jnp.float32).at[:n].set(x)
    x_aug = jnp.concatenate([x_p, onescol], axis=1)
    zf = jnp.zeros((WBC, da), jnp.float32)
    wl1t, wr1t = W1l.T, W1r.T
    wl2t, wr2t = W2l.T, W2r.T
    bl1 = b1l.reshape(1, d)
    bl2 = b2l.reshape(1, d)

    agg = _sc_agg_kernel(nr, da, nbatch)

    p1 = agg(x_aug, src_p, dst_p, wb, zf).reshape(NC, nr, da)
    h = _tc_layer(True, nr, d, p1, x_p, wl1t, bl1, wr1t)
    h_aug = jnp.concatenate([h, onescol], axis=1)
    p2 = agg(h_aug, src_p, dst_p, wb, zf).reshape(NC, nr, da)
    out = _tc_layer(False, nr, d, p2, h, wl2t, bl2, wr2t)
    return out[:n]


# speedup vs baseline: 7.8963x; 1.0090x over previous
import functools

import jax
import jax.numpy as jnp
from jax import lax
from jax.experimental import pallas as pl
from jax.experimental.pallas import tpu as pltpu
from jax.experimental.pallas import tpu_sc as plsc

NC = 2
NS = 16
NW = NC * NS
CHUNK = 128
GB = 4
ROWBLK = 1024
DPAD = 16
WBC = 128


def _sc_agg_kernel(nr, da, nbatch):
    rows_per_tile = nr // NS
    nzc = rows_per_tile // WBC
    assert rows_per_tile % WBC == 0
    mesh = plsc.VectorSubcoreMesh(
        core_axis_name="c", subcore_axis_name="s", num_cores=NC, num_subcores=NS
    )
    scratch = [
        pltpu.VMEM((GB, CHUNK), jnp.int32),
        pltpu.VMEM((GB, CHUNK), jnp.int32),
        pltpu.VMEM((CHUNK, da), jnp.float32),
        pltpu.VMEM((WBC, da), jnp.float32),
        pltpu.VMEM_SHARED((nr, da), jnp.float32),
        pltpu.SemaphoreType.DMA,
        pltpu.SemaphoreType.DMA,
    ]

    def body(feats, src_hbm, dst_hbm, wb_hbm, zf_hbm,
             sum_out, src_b, dst_b, rows0, stage_v, sum_sh,
             sem0, sem1):
        c = lax.axis_index("c")
        s = lax.axis_index("s")
        wid = c * NS + s
        row0 = s * rows_per_tile
        out0 = c * nr + row0
        bufs = (rows0, stage_v)
        sems = (sem0, sem1)

        wbrow = src_b.at[0]
        pltpu.sync_copy(zf_hbm, stage_v)
        for k in range(nzc):
            pltpu.sync_copy(wb_hbm.at[pl.ds(row0 + k * WBC, WBC)], wbrow)
            pltpu.sync_copy(stage_v, sum_sh.at[wbrow])
        plsc.subcore_barrier()

        def batch_body(b, carry):
            pltpu.sync_copy(src_hbm.at[wid * nbatch + b], src_b)
            pltpu.sync_copy(dst_hbm.at[wid * nbatch + b], dst_b)
            descs = [None, None]
            descs[0] = pltpu.async_copy(feats.at[src_b.at[0]], bufs[0],
                                        sems[0])
            for g in range(GB):
                cur = g % 2
                if g + 1 < GB:
                    nxt = 1 - cur
                    descs[nxt] = pltpu.async_copy(
                        feats.at[src_b.at[g + 1]], bufs[nxt], sems[nxt])
                descs[cur].wait()
                pltpu.sync_copy(bufs[cur], sum_sh.at[dst_b.at[g]], add=True)
            return carry

        lax.fori_loop(0, nbatch, batch_body, 0)
        plsc.subcore_barrier()

        for k in range(nzc):
            pltpu.sync_copy(wb_hbm.at[pl.ds(row0 + k * WBC, WBC)], wbrow)
            pltpu.async_copy(sum_sh.at[wbrow], stage_v, sem0).wait()
            pltpu.sync_copy(stage_v, sum_out.at[pl.ds(out0 + k * WBC, WBC)])

    return pl.kernel(
        body,
        out_type=jax.ShapeDtypeStruct((NC * nr, da), jnp.float32),
        mesh=mesh,
        scratch_types=scratch,
        compiler_params=pltpu.CompilerParams(use_tc_tiling_on_sc=False),
    )


def _tc_layer_kernel(relu, d, aug_out, p_ref, x_ref, wlt_ref, bl_ref,
                     wrt_ref, o_ref):
    psum = p_ref[0] + p_ref[1]
    summed = psum[:, :d]
    deg = psum[:, d:d + 1]
    mean = summed / jnp.maximum(deg, 1.0)
    h = (jnp.dot(mean, wlt_ref[...], preferred_element_type=jnp.float32)
         + bl_ref[...]
         + jnp.dot(x_ref[:, :d], wrt_ref[...],
                   preferred_element_type=jnp.float32))
    nrm = jnp.sqrt(jnp.sum(h * h, axis=1, keepdims=True))
    h = h / jnp.maximum(nrm, 1e-12)
    if relu:
        h = jnp.maximum(h, 0.0)
    if aug_out:
        h = jnp.concatenate(
            [h, jnp.ones((h.shape[0], DPAD), jnp.float32)], axis=1)
    o_ref[...] = h


def _tc_layer(relu, aug_out, nr, d, parts, x_aug, wlt, bl, wrt):
    nblk = nr // ROWBLK
    da = d + DPAD
    dout = da if aug_out else d
    return pl.pallas_call(
        functools.partial(_tc_layer_kernel, relu, d, aug_out),
        grid=(nblk,),
        in_specs=[
            pl.BlockSpec((NC, ROWBLK, da), lambda b: (0, b, 0)),
            pl.BlockSpec((ROWBLK, da), lambda b: (b, 0)),
            pl.BlockSpec((d, d), lambda b: (0, 0)),
            pl.BlockSpec((1, d), lambda b: (0, 0)),
            pl.BlockSpec((d, d), lambda b: (0, 0)),
        ],
        out_specs=pl.BlockSpec((ROWBLK, dout), lambda b: (b, 0)),
        out_shape=jax.ShapeDtypeStruct((nr, dout), jnp.float32),
    )(parts, x_aug, wlt, bl, wrt)


def kernel(x, edge_index, W1l, b1l, W1r, W2l, b2l, W2r):
    n, d = x.shape
    da = d + DPAD
    e = edge_index.shape[1]
    nr = ((n + ROWBLK - 1) // ROWBLK) * ROWBLK
    unit = NW * GB * CHUNK
    nbatch = -(-e // unit)
    ep = unit * nbatch

    src = edge_index[0].astype(jnp.int32)
    dst = edge_index[1].astype(jnp.int32)
    pad = ep - e
    pad_src = jnp.arange(pad, dtype=jnp.int32) % n
    pad_dst = n + jnp.arange(pad, dtype=jnp.int32) % (nr - n)
    src_p = jnp.concatenate([src, pad_src]).reshape(NW * nbatch, GB, CHUNK)
    dst_p = jnp.concatenate([dst, pad_dst]).reshape(NW * nbatch, GB, CHUNK)
    wb = jnp.arange(nr, dtype=jnp.int32)

    x_aug = jnp.zeros((nr, da), jnp.float32).at[:, d:].set(1.0)
    x_aug = x_aug.at[:n, :d].set(x)
    zf = jnp.zeros((WBC, da), jnp.float32)
    wl1t, wr1t = W1l.T, W1r.T
    wl2t, wr2t = W2l.T, W2r.T
    bl1 = b1l.reshape(1, d)
    bl2 = b2l.reshape(1, d)

    agg = _sc_agg_kernel(nr, da, nbatch)

    p1 = agg(x_aug, src_p, dst_p, wb, zf).reshape(NC, nr, da)
    h_aug = _tc_layer(True, True, nr, d, p1, x_aug, wl1t, bl1, wr1t)
    p2 = agg(h_aug, src_p, dst_p, wb, zf).reshape(NC, nr, da)
    out = _tc_layer(False, False, nr, d, p2, h_aug, wl2t, bl2, wr2t)
    return out[:n]


# speedup vs baseline: 7.9169x; 1.0026x over previous
import functools

import jax
import jax.numpy as jnp
from jax import lax
from jax.experimental import pallas as pl
from jax.experimental.pallas import tpu as pltpu
from jax.experimental.pallas import tpu_sc as plsc

NC = 2
NS = 16
NW = NC * NS
CHUNK = 128
GB = 4
ROWBLK = 1024
DPAD = 16
WBC = 128


def _sc_agg_kernel(nr, da, nbatch):
    rows_per_tile = nr // NS
    nzc = rows_per_tile // WBC
    assert rows_per_tile % WBC == 0
    mesh = plsc.VectorSubcoreMesh(
        core_axis_name="c", subcore_axis_name="s", num_cores=NC, num_subcores=NS
    )
    scratch = [
        pltpu.VMEM((GB, CHUNK), jnp.int32),
        pltpu.VMEM((GB, CHUNK), jnp.int32),
        pltpu.VMEM((CHUNK, da), jnp.float32),
        pltpu.VMEM((WBC, da), jnp.float32),
        pltpu.VMEM_SHARED((nr, da), jnp.float32),
        pltpu.SemaphoreType.DMA,
        pltpu.SemaphoreType.DMA,
        pltpu.SemaphoreType.DMA,
        pltpu.SemaphoreType.DMA,
    ]

    def body(feats, src_hbm, dst_hbm, wb_hbm, zf_hbm,
             sum_out, src_b, dst_b, rows0, stage_v, sum_sh,
             sem0, sem1, sem2, sem3):
        c = lax.axis_index("c")
        s = lax.axis_index("s")
        wid = c * NS + s
        row0 = s * rows_per_tile
        out0 = c * nr + row0
        bufs = (rows0, stage_v)
        gsems = (sem0, sem1)
        ssems = (sem2, sem3)

        wbrow = src_b.at[0]
        pltpu.sync_copy(zf_hbm, stage_v)
        for k in range(nzc):
            pltpu.sync_copy(wb_hbm.at[pl.ds(row0 + k * WBC, WBC)], wbrow)
            pltpu.sync_copy(stage_v, sum_sh.at[wbrow])
        plsc.subcore_barrier()

        def batch_body(b, carry):
            pltpu.sync_copy(src_hbm.at[wid * nbatch + b], src_b)
            pltpu.sync_copy(dst_hbm.at[wid * nbatch + b], dst_b)
            g_descs = [None, None]
            s_descs = [None, None]
            g_descs[0] = pltpu.async_copy(feats.at[src_b.at[0]], bufs[0],
                                          gsems[0])
            for g in range(GB):
                cur = g % 2
                if g + 1 < GB:
                    nxt = 1 - cur
                    if g >= 1:
                        s_descs[nxt].wait()
                    g_descs[nxt] = pltpu.async_copy(
                        feats.at[src_b.at[g + 1]], bufs[nxt], gsems[nxt])
                g_descs[cur].wait()
                s_descs[cur] = pltpu.async_copy(
                    bufs[cur], sum_sh.at[dst_b.at[g]], ssems[cur], add=True)
            s_descs[(GB - 2) % 2].wait()
            s_descs[(GB - 1) % 2].wait()
            return carry

        lax.fori_loop(0, nbatch, batch_body, 0)
        plsc.subcore_barrier()

        for k in range(nzc):
            pltpu.sync_copy(wb_hbm.at[pl.ds(row0 + k * WBC, WBC)], wbrow)
            pltpu.async_copy(sum_sh.at[wbrow], stage_v, sem0).wait()
            pltpu.sync_copy(stage_v, sum_out.at[pl.ds(out0 + k * WBC, WBC)])

    return pl.kernel(
        body,
        out_type=jax.ShapeDtypeStruct((NC * nr, da), jnp.float32),
        mesh=mesh,
        scratch_types=scratch,
        compiler_params=pltpu.CompilerParams(use_tc_tiling_on_sc=False),
    )


def _tc_layer_kernel(relu, d, aug_out, p_ref, x_ref, wlt_ref, bl_ref,
                     wrt_ref, o_ref):
    psum = p_ref[0] + p_ref[1]
    summed = psum[:, :d]
    deg = psum[:, d:d + 1]
    mean = summed / jnp.maximum(deg, 1.0)
    h = (jnp.dot(mean, wlt_ref[...], preferred_element_type=jnp.float32)
         + bl_ref[...]
         + jnp.dot(x_ref[:, :d], wrt_ref[...],
                   preferred_element_type=jnp.float32))
    nrm = jnp.sqrt(jnp.sum(h * h, axis=1, keepdims=True))
    h = h / jnp.maximum(nrm, 1e-12)
    if relu:
        h = jnp.maximum(h, 0.0)
    if aug_out:
        h = jnp.concatenate(
            [h, jnp.ones((h.shape[0], DPAD), jnp.float32)], axis=1)
    o_ref[...] = h


def _tc_layer(relu, aug_out, nr, d, parts, x_aug, wlt, bl, wrt):
    nblk = nr // ROWBLK
    da = d + DPAD
    dout = da if aug_out else d
    return pl.pallas_call(
        functools.partial(_tc_layer_kernel, relu, d, aug_out),
        grid=(nblk,),
        in_specs=[
            pl.BlockSpec((NC, ROWBLK, da), lambda b: (0, b, 0)),
            pl.BlockSpec((ROWBLK, da), lambda b: (b, 0)),
            pl.BlockSpec((d, d), lambda b: (0, 0)),
            pl.BlockSpec((1, d), lambda b: (0, 0)),
            pl.BlockSpec((d, d), lambda b: (0, 0)),
        ],
        out_specs=pl.BlockSpec((ROWBLK, dout), lambda b: (b, 0)),
        out_shape=jax.ShapeDtypeStruct((nr, dout), jnp.float32),
    )(parts, x_aug, wlt, bl, wrt)


def kernel(x, edge_index, W1l, b1l, W1r, W2l, b2l, W2r):
    n, d = x.shape
    da = d + DPAD
    e = edge_index.shape[1]
    nr = ((n + ROWBLK - 1) // ROWBLK) * ROWBLK
    unit = NW * GB * CHUNK
    nbatch = -(-e // unit)
    ep = unit * nbatch

    src = edge_index[0].astype(jnp.int32)
    dst = edge_index[1].astype(jnp.int32)
    pad = ep - e
    pad_src = jnp.arange(pad, dtype=jnp.int32) % n
    pad_dst = n + jnp.arange(pad, dtype=jnp.int32) % (nr - n)
    src_p = jnp.concatenate([src, pad_src]).reshape(NW * nbatch, GB, CHUNK)
    dst_p = jnp.concatenate([dst, pad_dst]).reshape(NW * nbatch, GB, CHUNK)
    wb = jnp.arange(nr, dtype=jnp.int32)

    x_aug = jnp.zeros((nr, da), jnp.float32).at[:, d:].set(1.0)
    x_aug = x_aug.at[:n, :d].set(x)
    zf = jnp.zeros((WBC, da), jnp.float32)
    wl1t, wr1t = W1l.T, W1r.T
    wl2t, wr2t = W2l.T, W2r.T
    bl1 = b1l.reshape(1, d)
    bl2 = b2l.reshape(1, d)

    agg = _sc_agg_kernel(nr, da, nbatch)

    p1 = agg(x_aug, src_p, dst_p, wb, zf).reshape(NC, nr, da)
    h_aug = _tc_layer(True, True, nr, d, p1, x_aug, wl1t, bl1, wr1t)
    p2 = agg(h_aug, src_p, dst_p, wb, zf).reshape(NC, nr, da)
    out = _tc_layer(False, False, nr, d, p2, h_aug, wl2t, bl2, wr2t)
    return out[:n]
